# baseline (speedup 1.0000x reference)
import functools

import jax
import jax.numpy as jnp
from jax import lax
from jax.experimental import pallas as pl
from jax.experimental.pallas import tpu as pltpu
from jax.experimental.pallas import tpu_sc as plsc

_N = 10000
_E = 160000
_D_IN = 256
_D_H = 512
_D_OUT = 256

_NSUB = 16
_NCORE = 2
_NW = _NSUB * _NCORE
_NPAD = 10240
_RPW = _NPAD // _NW
_ACC_R = _RPW + 1
_K = 48
_CAP = 5760
_NCHUNK = _CAP // _K
_NPAIR = _NCHUNK // 2
_CAPM = _CAP + 16
_SK = 1024
_NSCAN = 158
_EPAD = _SK * _NSCAN
_NSPAIR = _NSCAN // 2


def _partition():
  mesh = plsc.VectorSubcoreMesh(core_axis_name="core", subcore_axis_name="subcore")
  out_type = [jax.ShapeDtypeStruct((_NW * _CAP,), jnp.int32),
              jax.ShapeDtypeStruct((_NW * _CAP,), jnp.int32),
              jax.ShapeDtypeStruct((_NW * 16,), jnp.int32)]
  scratch = [
      pltpu.VMEM((_SK,), jnp.int32),
      pltpu.VMEM((_SK,), jnp.int32),
      pltpu.VMEM((_SK,), jnp.int32),
      pltpu.VMEM((_SK,), jnp.int32),
      pltpu.VMEM((_CAPM,), jnp.int32),
      pltpu.VMEM((_CAPM,), jnp.int32),
      pltpu.VMEM((16,), jnp.int32),
      pltpu.SemaphoreType.DMA,
      pltpu.SemaphoreType.DMA,
  ]

  def body(src_h, dst_h, slist_o, dlist_o, cnt_o, sstage0, dstage0, sstage1,
           dstage1, slist, dlist, cstage, sem0, sem1):
    c = lax.axis_index("core")
    s = lax.axis_index("subcore")
    w = c * _NSUB + s
    lo = w * _RPW
    hi = lo + _RPW

    pad_s = jnp.zeros((16,), jnp.int32)
    pad_d = jnp.full((16,), _RPW, jnp.int32)

    @pl.loop(0, _CAPM // 16)
    def _(i):
      slist[pl.ds(i * 16, 16)] = pad_s
      dlist[pl.ds(i * 16, 16)] = pad_d

    def stage(ci, sstage, dstage, sem):
      pltpu.async_copy(src_h.at[pl.ds(ci * _SK, _SK)], sstage, sem)
      pltpu.async_copy(dst_h.at[pl.ds(ci * _SK, _SK)], dstage, sem)

    def wait(ci, sstage, dstage, sem):
      pltpu.make_async_copy(src_h.at[pl.ds(ci * _SK, _SK)], sstage, sem).wait()
      pltpu.make_async_copy(dst_h.at[pl.ds(ci * _SK, _SK)], dstage, sem).wait()

    def scan(sstage, dstage, off):
      for j in range(_SK // 16):
        dv = dstage[pl.ds(j * 16, 16)]
        sv = sstage[pl.ds(j * 16, 16)]
        m = (dv >= lo) & (dv < hi)
        offc = jnp.minimum(off, _CAP)
        plsc.store_compressed(slist.at[pl.ds(offc, 16)], sv, mask=m)
        plsc.store_compressed(dlist.at[pl.ds(offc, 16)], dv - lo, mask=m)
        off = off + plsc.all_reduce_population_count(m)[0]
      return off

    stage(0, sstage0, dstage0, sem0)

    def pair(pi, off):
      i0 = 2 * pi
      wait(i0, sstage0, dstage0, sem0)
      stage(i0 + 1, sstage1, dstage1, sem1)
      off = scan(sstage0, dstage0, off)
      wait(i0 + 1, sstage1, dstage1, sem1)

      @pl.when(pi < _NSPAIR - 1)
      def _():
        stage(i0 + 2, sstage0, dstage0, sem0)

      return scan(sstage1, dstage1, off)

    total = lax.fori_loop(0, _NSPAIR, pair, jnp.int32(0))
    cstage[pl.ds(0, 16)] = jnp.full((16,), total, jnp.int32)
    pltpu.sync_copy(cstage, cnt_o.at[pl.ds(w * 16, 16)])
    pltpu.sync_copy(slist.at[pl.ds(0, _CAP)], slist_o.at[pl.ds(w * _CAP, _CAP)])
    pltpu.sync_copy(dlist.at[pl.ds(0, _CAP)], dlist_o.at[pl.ds(w * _CAP, _CAP)])

  cp = pltpu.CompilerParams(needs_layout_passes=False)
  return pl.kernel(body, out_type=out_type, mesh=mesh, scratch_types=scratch,
                   compiler_params=cp)


def _make_agg(d, with_deg):
  mesh = plsc.VectorSubcoreMesh(core_axis_name="core", subcore_axis_name="subcore")
  out_type = [jax.ShapeDtypeStruct((_NPAD * d,), jnp.float32)]
  if with_deg:
    out_type.append(jax.ShapeDtypeStruct((_NPAD * 16,), jnp.float32))
  scratch = [
      pltpu.VMEM((_CAP,), jnp.int32),
      pltpu.VMEM((_CAPM,), jnp.int32),
      pltpu.VMEM((_K, d), jnp.float32),
      pltpu.VMEM((_K, d), jnp.float32),
      pltpu.VMEM((_ACC_R * d,), jnp.float32),
      pltpu.VMEM((16,), jnp.int32),
      pltpu.SemaphoreType.DMA,
      pltpu.SemaphoreType.DMA,
  ]
  if with_deg:
    scratch.append(pltpu.VMEM((_ACC_R * 16,), jnp.float32))

  def body(*refs):
    if with_deg:
      (table, slist_h, dlist_h, cnt_h, zacc, z16, agg_o, deg_o,
       sl_v, dl_v, rows0, rows1, acc, cnt_v, sem0, sem1, dacc) = refs
    else:
      (table, slist_h, dlist_h, cnt_h, zacc, agg_o,
       sl_v, dl_v, rows0, rows1, acc, cnt_v, sem0, sem1) = refs
    c = lax.axis_index("core")
    s = lax.axis_index("subcore")
    w = c * _NSUB + s

    pltpu.sync_copy(slist_h.at[pl.ds(w * _CAP, _CAP)], sl_v)
    pltpu.sync_copy(dlist_h.at[pl.ds(w * _CAP, _CAP)], dl_v.at[pl.ds(0, _CAP)])
    pltpu.sync_copy(cnt_h.at[pl.ds(w * 16, 16)], cnt_v)
    my_cnt = cnt_v[pl.ds(0, 16)][0]
    my_npair = (my_cnt + 2 * _K - 1) // (2 * _K)
    pltpu.sync_copy(zacc, acc)
    if with_deg:
      pltpu.sync_copy(z16, dacc)
    ones16 = jnp.ones((16,), jnp.float32)

    def gather(ci, rows, sem):
      pltpu.async_copy(table.at[sl_v.at[pl.ds(ci * _K, _K)]], rows, sem)

    def gwait(ci, rows, sem):
      pltpu.make_async_copy(table.at[sl_v.at[pl.ds(ci * _K, _K)]], rows,
                            sem).wait()

    def compute(ci, rows):
      @plsc.parallel_loop(0, _K, unroll=8)
      def _(e):
        dl = dl_v[pl.ds(ci * _K + e, 16)][0]
        for j in range(d // 16):
          plsc.addupdate(acc.at[pl.ds(dl * d + j * 16, 16)],
                         rows[e, pl.ds(j * 16, 16)])
        if with_deg:
          plsc.addupdate(dacc.at[pl.ds(dl * 16, 16)], ones16)

    gather(0, rows0, sem0)

    @pl.loop(0, my_npair)
    def _(pi):
      i0 = 2 * pi
      gwait(i0, rows0, sem0)
      gather(i0 + 1, rows1, sem1)
      compute(i0, rows0)
      gwait(i0 + 1, rows1, sem1)

      @pl.when(pi < my_npair - 1)
      def _():
        gather(i0 + 2, rows0, sem0)

      compute(i0 + 1, rows1)

    pltpu.sync_copy(acc.at[pl.ds(0, _RPW * d)],
                    agg_o.at[pl.ds(w * _RPW * d, _RPW * d)])
    if with_deg:
      pltpu.sync_copy(dacc.at[pl.ds(0, _RPW * 16)],
                      deg_o.at[pl.ds(w * _RPW * 16, _RPW * 16)])

  cp = pltpu.CompilerParams(needs_layout_passes=False)
  return pl.kernel(body, out_type=out_type, mesh=mesh, scratch_types=scratch,
                   compiler_params=cp)


def _tc1_body(agg, deg, x, w1l, w1r, b1, w2l, h_o, p_o):
  inv = 1.0 / jnp.maximum(deg[...][:, :1], 1.0)
  mean = agg[...] * inv
  pre = (jnp.dot(mean, w1l[...], preferred_element_type=jnp.float32)
         + jnp.dot(x[...], w1r[...], preferred_element_type=jnp.float32)
         + b1[...])
  h = jnp.maximum(pre, 0.0)
  h_o[...] = h
  p_o[...] = jnp.dot(h, w2l[...], preferred_element_type=jnp.float32)


def _tc2_body(agg, deg, h, w2r, b2, o):
  inv = 1.0 / jnp.maximum(deg[...][:, :1], 1.0)
  pre = (agg[...] * inv
         + jnp.dot(h[...], w2r[...], preferred_element_type=jnp.float32)
         + b2[...])
  m = jnp.max(pre, axis=1, keepdims=True)
  e = jnp.exp(pre - m)
  lse = jnp.log(jnp.sum(e, axis=1, keepdims=True))
  o[...] = pre - m - lse


_RB = 640


def _tc1(agg, deg, x, w1l, w1r, b1, w2l):
  nb = _NPAD // _RB
  return pl.pallas_call(
      _tc1_body,
      grid=(nb,),
      in_specs=[
          pl.BlockSpec((_RB, _D_IN), lambda i: (i, 0)),
          pl.BlockSpec((_RB, 16), lambda i: (i, 0)),
          pl.BlockSpec((_RB, _D_IN), lambda i: (i, 0)),
          pl.BlockSpec((_D_IN, _D_H), lambda i: (0, 0)),
          pl.BlockSpec((_D_IN, _D_H), lambda i: (0, 0)),
          pl.BlockSpec((1, _D_H), lambda i: (0, 0)),
          pl.BlockSpec((_D_H, _D_OUT), lambda i: (0, 0)),
      ],
      out_specs=[
          pl.BlockSpec((_RB, _D_H), lambda i: (i, 0)),
          pl.BlockSpec((_RB, _D_OUT), lambda i: (i, 0)),
      ],
      out_shape=[jax.ShapeDtypeStruct((_NPAD, _D_H), jnp.float32),
                 jax.ShapeDtypeStruct((_NPAD, _D_OUT), jnp.float32)],
  )(agg, deg, x, w1l, w1r, b1, w2l)


def _tc2(agg, deg, h, w2r, b2):
  nb = _NPAD // _RB
  return pl.pallas_call(
      _tc2_body,
      grid=(nb,),
      in_specs=[
          pl.BlockSpec((_RB, _D_OUT), lambda i: (i, 0)),
          pl.BlockSpec((_RB, 16), lambda i: (i, 0)),
          pl.BlockSpec((_RB, _D_H), lambda i: (i, 0)),
          pl.BlockSpec((_D_H, _D_OUT), lambda i: (0, 0)),
          pl.BlockSpec((1, _D_OUT), lambda i: (0, 0)),
      ],
      out_specs=pl.BlockSpec((_RB, _D_OUT), lambda i: (i, 0)),
      out_shape=jax.ShapeDtypeStruct((_NPAD, _D_OUT), jnp.float32),
  )(agg, deg, h, w2r, b2)


def kernel(x, edge_index, W1l, W1r, b1, W2l, W2r, b2):
  src = jnp.pad(edge_index[0], (0, _EPAD - _E), constant_values=0)
  dst = jnp.pad(edge_index[1], (0, _EPAD - _E), constant_values=-1)
  x_pad = jnp.pad(x, ((0, _NPAD - _N), (0, 0)))
  zacc = jnp.zeros((_ACC_R * _D_IN,), jnp.float32)
  z16 = jnp.zeros((_ACC_R * 16,), jnp.float32)

  slist, dlist, cnts = _partition()(src, dst)
  agg1, deg16 = _make_agg(_D_IN, True)(x_pad, slist, dlist, cnts, zacc, z16)
  agg1 = agg1.reshape(_NPAD, _D_IN)
  deg16 = deg16.reshape(_NPAD, 16)
  h, p = _tc1(agg1, deg16, x_pad, W1l, W1r, b1.reshape(1, _D_H), W2l)
  (agg2,) = _make_agg(_D_OUT, False)(p, slist, dlist, cnts, zacc)
  agg2 = agg2.reshape(_NPAD, _D_OUT)
  out = _tc2(agg2, deg16, h, W2r, b2.reshape(1, _D_OUT))
  return out[:_N]


# speedup vs baseline: 1.0299x; 1.0299x over previous
import functools

import jax
import jax.numpy as jnp
from jax import lax
from jax.experimental import pallas as pl
from jax.experimental.pallas import tpu as pltpu
from jax.experimental.pallas import tpu_sc as plsc

_N = 10000
_E = 160000
_D_IN = 256
_D_H = 512
_D_OUT = 256

_NSUB = 16
_NCORE = 2
_NW = _NSUB * _NCORE
_NPAD = 10240
_RPW = _NPAD // _NW
_ACC_R = _RPW + 1
_K = 48
_CAP = 5760
_NCHUNK = _CAP // _K
_NPAIR = _NCHUNK // 2
_CAPM = _CAP + 16
_SK = 1024
_NSCAN = 158
_EPAD = _SK * _NSCAN
_NSPAIR = _NSCAN // 2


def _partition():
  mesh = plsc.VectorSubcoreMesh(core_axis_name="core", subcore_axis_name="subcore")
  out_type = [jax.ShapeDtypeStruct((_NW * _CAP,), jnp.int32),
              jax.ShapeDtypeStruct((_NW * _CAP,), jnp.int32),
              jax.ShapeDtypeStruct((_NW * 16,), jnp.int32)]
  scratch = [
      pltpu.VMEM((_SK,), jnp.int32),
      pltpu.VMEM((_SK,), jnp.int32),
      pltpu.VMEM((_SK,), jnp.int32),
      pltpu.VMEM((_SK,), jnp.int32),
      pltpu.VMEM((_CAPM,), jnp.int32),
      pltpu.VMEM((_CAPM,), jnp.int32),
      pltpu.VMEM((16,), jnp.int32),
      pltpu.SemaphoreType.DMA,
      pltpu.SemaphoreType.DMA,
  ]

  def body(src_h, dst_h, slist_o, dlist_o, cnt_o, sstage0, dstage0, sstage1,
           dstage1, slist, dlist, cstage, sem0, sem1):
    c = lax.axis_index("core")
    s = lax.axis_index("subcore")
    w = c * _NSUB + s
    lo = w * _RPW
    hi = lo + _RPW

    pad_s = jnp.zeros((16,), jnp.int32)
    pad_d = jnp.full((16,), _RPW, jnp.int32)

    @pl.loop(0, _CAPM // 16)
    def _(i):
      slist[pl.ds(i * 16, 16)] = pad_s
      dlist[pl.ds(i * 16, 16)] = pad_d

    def stage(ci, sstage, dstage, sem):
      pltpu.async_copy(src_h.at[pl.ds(ci * _SK, _SK)], sstage, sem)
      pltpu.async_copy(dst_h.at[pl.ds(ci * _SK, _SK)], dstage, sem)

    def wait(ci, sstage, dstage, sem):
      pltpu.make_async_copy(src_h.at[pl.ds(ci * _SK, _SK)], sstage, sem).wait()
      pltpu.make_async_copy(dst_h.at[pl.ds(ci * _SK, _SK)], dstage, sem).wait()

    def scan(sstage, dstage, off):
      for j in range(_SK // 16):
        dv = dstage[pl.ds(j * 16, 16)]
        sv = sstage[pl.ds(j * 16, 16)]
        m = (dv >= lo) & (dv < hi)
        offc = jnp.minimum(off, _CAP)
        plsc.store_compressed(slist.at[pl.ds(offc, 16)], sv, mask=m)
        plsc.store_compressed(dlist.at[pl.ds(offc, 16)], dv - lo, mask=m)
        off = off + plsc.all_reduce_population_count(m)[0]
      return off

    stage(0, sstage0, dstage0, sem0)

    def pair(pi, off):
      i0 = 2 * pi
      wait(i0, sstage0, dstage0, sem0)
      stage(i0 + 1, sstage1, dstage1, sem1)
      off = scan(sstage0, dstage0, off)
      wait(i0 + 1, sstage1, dstage1, sem1)

      @pl.when(pi < _NSPAIR - 1)
      def _():
        stage(i0 + 2, sstage0, dstage0, sem0)

      return scan(sstage1, dstage1, off)

    total = lax.fori_loop(0, _NSPAIR, pair, jnp.int32(0))
    cstage[pl.ds(0, 16)] = jnp.full((16,), total, jnp.int32)
    pltpu.sync_copy(cstage, cnt_o.at[pl.ds(w * 16, 16)])
    pltpu.sync_copy(slist.at[pl.ds(0, _CAP)], slist_o.at[pl.ds(w * _CAP, _CAP)])
    pltpu.sync_copy(dlist.at[pl.ds(0, _CAP)], dlist_o.at[pl.ds(w * _CAP, _CAP)])

  cp = pltpu.CompilerParams(needs_layout_passes=False)
  return pl.kernel(body, out_type=out_type, mesh=mesh, scratch_types=scratch,
                   compiler_params=cp)


def _make_agg(d, with_deg):
  mesh = plsc.VectorSubcoreMesh(core_axis_name="core", subcore_axis_name="subcore")
  out_type = [jax.ShapeDtypeStruct((_NPAD * d,), jnp.float32)]
  if with_deg:
    out_type.append(jax.ShapeDtypeStruct((_NPAD * 16,), jnp.float32))
  scratch = [
      pltpu.VMEM((_CAP,), jnp.int32),
      pltpu.VMEM((_CAPM,), jnp.int32),
      pltpu.VMEM((_K, d), jnp.float32),
      pltpu.VMEM((_K, d), jnp.float32),
      pltpu.VMEM((_ACC_R * d,), jnp.float32),
      pltpu.VMEM((16,), jnp.int32),
      pltpu.SemaphoreType.DMA,
      pltpu.SemaphoreType.DMA,
  ]
  if with_deg:
    scratch.append(pltpu.VMEM((_ACC_R * 16,), jnp.float32))

  def body(*refs):
    if with_deg:
      (table, slist_h, dlist_h, cnt_h, zacc, z16, agg_o, deg_o,
       sl_v, dl_v, rows0, rows1, acc, cnt_v, sem0, sem1, dacc) = refs
    else:
      (table, slist_h, dlist_h, cnt_h, zacc, agg_o,
       sl_v, dl_v, rows0, rows1, acc, cnt_v, sem0, sem1) = refs
    c = lax.axis_index("core")
    s = lax.axis_index("subcore")
    w = c * _NSUB + s

    pltpu.sync_copy(slist_h.at[pl.ds(w * _CAP, _CAP)], sl_v)
    pltpu.sync_copy(dlist_h.at[pl.ds(w * _CAP, _CAP)], dl_v.at[pl.ds(0, _CAP)])
    pltpu.sync_copy(cnt_h.at[pl.ds(w * 16, 16)], cnt_v)
    my_cnt = cnt_v[pl.ds(0, 16)][0]
    my_npair = (my_cnt + 2 * _K - 1) // (2 * _K)
    pltpu.sync_copy(zacc, acc)
    if with_deg:
      pltpu.sync_copy(z16, dacc)
    ones16 = jnp.ones((16,), jnp.float32)

    def gather(ci, rows, sem):
      pltpu.async_copy(table.at[sl_v.at[pl.ds(ci * _K, _K)]], rows, sem)

    def gwait(ci, rows, sem):
      pltpu.make_async_copy(table.at[sl_v.at[pl.ds(ci * _K, _K)]], rows,
                            sem).wait()

    def compute(ci, rows):
      @plsc.parallel_loop(0, _K, unroll=4)
      def _(e):
        dl = dl_v[pl.ds(ci * _K + e, 16)][0]
        for j in range(d // 16):
          plsc.addupdate(acc.at[pl.ds(dl * d + j * 16, 16)],
                         rows[e, pl.ds(j * 16, 16)])
        if with_deg:
          plsc.addupdate(dacc.at[pl.ds(dl * 16, 16)], ones16)

    gather(0, rows0, sem0)
    gather(1, rows1, sem1)

    @pl.loop(0, my_npair)
    def _(pi):
      i0 = 2 * pi
      gwait(i0, rows0, sem0)
      compute(i0, rows0)

      @pl.when(pi < my_npair - 1)
      def _():
        gather(i0 + 2, rows0, sem0)

      gwait(i0 + 1, rows1, sem1)
      compute(i0 + 1, rows1)

      @pl.when(pi < my_npair - 1)
      def _():
        gather(i0 + 3, rows1, sem1)

    pltpu.sync_copy(acc.at[pl.ds(0, _RPW * d)],
                    agg_o.at[pl.ds(w * _RPW * d, _RPW * d)])
    if with_deg:
      pltpu.sync_copy(dacc.at[pl.ds(0, _RPW * 16)],
                      deg_o.at[pl.ds(w * _RPW * 16, _RPW * 16)])

  cp = pltpu.CompilerParams(needs_layout_passes=False)
  return pl.kernel(body, out_type=out_type, mesh=mesh, scratch_types=scratch,
                   compiler_params=cp)


def _tc1_body(agg, deg, x, w1l, w1r, b1, w2l, h_o, p_o):
  inv = 1.0 / jnp.maximum(deg[...][:, :1], 1.0)
  mean = agg[...] * inv
  pre = (jnp.dot(mean, w1l[...], preferred_element_type=jnp.float32)
         + jnp.dot(x[...], w1r[...], preferred_element_type=jnp.float32)
         + b1[...])
  h = jnp.maximum(pre, 0.0)
  h_o[...] = h
  p_o[...] = jnp.dot(h, w2l[...], preferred_element_type=jnp.float32)


def _tc2_body(agg, deg, h, w2r, b2, o):
  inv = 1.0 / jnp.maximum(deg[...][:, :1], 1.0)
  pre = (agg[...] * inv
         + jnp.dot(h[...], w2r[...], preferred_element_type=jnp.float32)
         + b2[...])
  m = jnp.max(pre, axis=1, keepdims=True)
  e = jnp.exp(pre - m)
  lse = jnp.log(jnp.sum(e, axis=1, keepdims=True))
  o[...] = pre - m - lse


_RB = 640


def _tc1(agg, deg, x, w1l, w1r, b1, w2l):
  nb = _NPAD // _RB
  return pl.pallas_call(
      _tc1_body,
      grid=(nb,),
      in_specs=[
          pl.BlockSpec((_RB, _D_IN), lambda i: (i, 0)),
          pl.BlockSpec((_RB, 16), lambda i: (i, 0)),
          pl.BlockSpec((_RB, _D_IN), lambda i: (i, 0)),
          pl.BlockSpec((_D_IN, _D_H), lambda i: (0, 0)),
          pl.BlockSpec((_D_IN, _D_H), lambda i: (0, 0)),
          pl.BlockSpec((1, _D_H), lambda i: (0, 0)),
          pl.BlockSpec((_D_H, _D_OUT), lambda i: (0, 0)),
      ],
      out_specs=[
          pl.BlockSpec((_RB, _D_H), lambda i: (i, 0)),
          pl.BlockSpec((_RB, _D_OUT), lambda i: (i, 0)),
      ],
      out_shape=[jax.ShapeDtypeStruct((_NPAD, _D_H), jnp.float32),
                 jax.ShapeDtypeStruct((_NPAD, _D_OUT), jnp.float32)],
  )(agg, deg, x, w1l, w1r, b1, w2l)


def _tc2(agg, deg, h, w2r, b2):
  nb = _NPAD // _RB
  return pl.pallas_call(
      _tc2_body,
      grid=(nb,),
      in_specs=[
          pl.BlockSpec((_RB, _D_OUT), lambda i: (i, 0)),
          pl.BlockSpec((_RB, 16), lambda i: (i, 0)),
          pl.BlockSpec((_RB, _D_H), lambda i: (i, 0)),
          pl.BlockSpec((_D_H, _D_OUT), lambda i: (0, 0)),
          pl.BlockSpec((1, _D_OUT), lambda i: (0, 0)),
      ],
      out_specs=pl.BlockSpec((_RB, _D_OUT), lambda i: (i, 0)),
      out_shape=jax.ShapeDtypeStruct((_NPAD, _D_OUT), jnp.float32),
  )(agg, deg, h, w2r, b2)


def kernel(x, edge_index, W1l, W1r, b1, W2l, W2r, b2):
  src = jnp.pad(edge_index[0], (0, _EPAD - _E), constant_values=0)
  dst = jnp.pad(edge_index[1], (0, _EPAD - _E), constant_values=-1)
  x_pad = jnp.pad(x, ((0, _NPAD - _N), (0, 0)))
  zacc = jnp.zeros((_ACC_R * _D_IN,), jnp.float32)
  z16 = jnp.zeros((_ACC_R * 16,), jnp.float32)

  slist, dlist, cnts = _partition()(src, dst)
  agg1, deg16 = _make_agg(_D_IN, True)(x_pad, slist, dlist, cnts, zacc, z16)
  agg1 = agg1.reshape(_NPAD, _D_IN)
  deg16 = deg16.reshape(_NPAD, 16)
  h, p = _tc1(agg1, deg16, x_pad, W1l, W1r, b1.reshape(1, _D_H), W2l)
  (agg2,) = _make_agg(_D_OUT, False)(p, slist, dlist, cnts, zacc)
  agg2 = agg2.reshape(_NPAD, _D_OUT)
  out = _tc2(agg2, deg16, h, W2r, b2.reshape(1, _D_OUT))
  return out[:_N]
